# baseline (speedup 1.0000x reference)
import dataclasses
import functools

import jax
from jax import lax
import jax.numpy as jnp
from jax.experimental import pallas as pl
from jax.experimental.pallas import tpu as pltpu
from jax.experimental.pallas import tpu_sc as plsc

K = 4
BS = 32
LEN_A = 4096
LEN_B = 64
LEN_DST = LEN_A + LEN_B
NC = 2
NS = 16
L = 16


def _sc_kernel(dst_hbm, a_hbm, b_hbm, la_hbm, lb_hbm, out_hbm,
               av, bv, dv, lav, lbv):
    g = lax.axis_index("s") * NC + lax.axis_index("c")
    pltpu.sync_copy(a_hbm.at[g], av.at[pl.ds(0, LEN_A)])
    pltpu.sync_copy(b_hbm.at[pl.ds(g * K, K)], bv)
    pltpu.sync_copy(la_hbm, lav)
    pltpu.sync_copy(lb_hbm, lbv)

    gvec = jnp.full((L,), g, jnp.int32)
    la_s = jnp.max(plsc.load_gather(lav, [gvec]))
    iota = lax.iota(jnp.int32, L)
    n64 = la_s // 64
    s0 = n64 * 64

    for r in range(K):
        row = g * K + r
        lb_s = jnp.max(plsc.load_gather(lbv, [gvec * K + r]))
        pltpu.sync_copy(dst_hbm.at[row], dv)

        @pl.loop(0, n64)
        def _(i):
            base = i * 64
            for t in range(4):
                dv[pl.ds(base + t * 16, 16)] = av[pl.ds(base + t * 16, 16)]

        rvec = jnp.full((L,), r, jnp.int32)
        for t in range(8):
            base = s0 + t * 16
            col = iota + base
            bval = plsc.load_gather(
                bv, [rvec, jnp.clip(col - la_s, 0, LEN_B - 1)])
            cur = dv[pl.ds(base, 16)]
            aval = av[pl.ds(base, 16)]
            dv[pl.ds(base, 16)] = jnp.where(
                col < la_s, aval,
                jnp.where(col < la_s + lb_s, bval, cur))

        pltpu.sync_copy(dv, out_hbm.at[row])


def kernel(page_table_dst, page_table_a, page_table_b, seq_len_a, seq_len_b):
    mesh = plsc.VectorSubcoreMesh(core_axis_name="c", subcore_axis_name="s")
    cp = pltpu.CompilerParams()
    if "needs_layout_passes" in pltpu.CompilerParams.__dataclass_fields__:
        cp = dataclasses.replace(cp, needs_layout_passes=False)
    run = functools.partial(
        pl.kernel,
        mesh=mesh,
        compiler_params=cp,
        out_type=jax.ShapeDtypeStruct(page_table_dst.shape,
                                      page_table_dst.dtype),
        scratch_types=[
            pltpu.VMEM((LEN_DST,), jnp.float32),
            pltpu.VMEM((K, LEN_B), jnp.float32),
            pltpu.VMEM((LEN_DST,), jnp.float32),
            pltpu.VMEM((BS,), jnp.int32),
            pltpu.VMEM((BS * K,), jnp.int32),
        ],
    )(_sc_kernel)
    return run(page_table_dst, page_table_a, page_table_b,
               seq_len_a.astype(jnp.int32), seq_len_b.astype(jnp.int32))


# speedup vs baseline: 1.0916x; 1.0916x over previous
import dataclasses
import functools

import jax
from jax import lax
import jax.numpy as jnp
from jax.experimental import pallas as pl
from jax.experimental.pallas import tpu as pltpu
from jax.experimental.pallas import tpu_sc as plsc

K = 4
BS = 32
LEN_A = 4096
LEN_B = 64
LEN_DST = LEN_A + LEN_B
TAIL = LEN_DST - 128
NC = 2
L = 16
SIZES = (2048, 1024, 512, 256, 128, 64)


def _sc_kernel(dst_hbm, a_hbm, b_hbm, la_hbm, lb_hbm, out_hbm,
               av, bv, dv0, dv1, dv2, dv3, dw0, dw1, dw2, dw3,
               wv0, wv1, wv2, wv3, lav, lbv,
               semA, semL, semB, semR0, semR1, semR2, semR3,
               semO0, semO1, semO2, semO3):
    semR = (semR0, semR1, semR2, semR3)
    semO = (semO0, semO1, semO2, semO3)
    dv = (dv0, dv1, dv2, dv3)
    dw = (dw0, dw1, dw2, dw3)
    wv = (wv0, wv1, wv2, wv3)
    g = lax.axis_index("s") * NC + lax.axis_index("c")
    pltpu.sync_copy(la_hbm, lav)
    pltpu.sync_copy(lb_hbm, lbv)

    gvec = jnp.full((L,), g, jnp.int32)
    la_s = jnp.max(plsc.load_gather(lav, [gvec]))
    s0 = pl.multiple_of((la_s // 64) * 64, 64)
    iota = lax.iota(jnp.int32, L)

    h_av = pltpu.async_copy(a_hbm.at[pl.ds(g * LEN_A, LEN_A)], av, semA)
    h_bv = pltpu.async_copy(b_hbm.at[pl.ds(g * K * LEN_B, K * LEN_B)],
                            bv, semA)

    rbase = [(g * K + r) * LEN_DST for r in range(K)]

    h_in, h_dw = [], []
    for r in range(K):
        h_in.append(pltpu.async_copy(
            dst_hbm.at[pl.ds(rbase[r] + 128, TAIL)], dv[r], semR[r]))
        h_dw.append(pltpu.async_copy(
            dst_hbm.at[pl.ds(rbase[r] + s0, 128)], dw[r], semR[r]))

    h_out = []
    for r in range(K):
        h_in[r].wait()
        h_dw[r].wait()
        h_out.append(pltpu.async_copy(
            dv[r], out_hbm.at[pl.ds(rbase[r] + 128, TAIL)], semO[r]))

    h_av.wait()
    h_bv.wait()
    for r in range(K):
        lb_s = jnp.max(plsc.load_gather(lbv, [gvec * K + r]))
        lab = la_s + lb_s
        for t in range(8):
            base = s0 + t * 16
            col = iota + base
            bval = plsc.load_gather(
                bv, [r * LEN_B + jnp.clip(col - la_s, 0, LEN_B - 1)])
            cur = dw[r][pl.ds(t * 16, 16)]
            sel = jnp.where(col < lab, bval, cur)
            if t < 4:
                aval = av[pl.ds(base, 16)]
                sel = jnp.where(col < la_s, aval, sel)
            wv[r][pl.ds(t * 16, 16)] = sel

    for r in range(K):
        h_out[r].wait()
        for size in SIZES:
            off = pl.multiple_of(s0 & (2 * LEN_A - 2 * size), 2 * size)

            @pl.when((la_s & size) != 0)
            def _(off=off, size=size, r=r):
                pltpu.async_copy(
                    av.at[pl.ds(off, size)],
                    out_hbm.at[pl.ds(rbase[r] + off, size)], semL)

        pltpu.async_copy(wv[r], out_hbm.at[pl.ds(rbase[r] + s0, 128)],
                         semB)

    for size in SIZES:

        @pl.when((la_s & size) != 0)
        def _(size=size):
            for _ in range(K):
                pltpu.make_async_copy(
                    dst_hbm.at[pl.ds(0, size)], av.at[pl.ds(0, size)],
                    semL).wait()

    for r in range(K):
        pltpu.make_async_copy(dst_hbm.at[pl.ds(0, 128)], wv[r],
                              semB).wait()


def kernel(page_table_dst, page_table_a, page_table_b, seq_len_a, seq_len_b):
    mesh = plsc.VectorSubcoreMesh(core_axis_name="c", subcore_axis_name="s")
    cp = pltpu.CompilerParams()
    if "needs_layout_passes" in pltpu.CompilerParams.__dataclass_fields__:
        cp = dataclasses.replace(cp, needs_layout_passes=False)
    run = functools.partial(
        pl.kernel,
        mesh=mesh,
        compiler_params=cp,
        out_type=jax.ShapeDtypeStruct((page_table_dst.size,),
                                      page_table_dst.dtype),
        scratch_types=[
            pltpu.VMEM((LEN_A,), jnp.float32),
            pltpu.VMEM((K * LEN_B,), jnp.float32),
        ] + [pltpu.VMEM((TAIL,), jnp.float32)] * K
          + [pltpu.VMEM((128,), jnp.float32)] * K
          + [pltpu.VMEM((128,), jnp.float32)] * K
          + [
            pltpu.VMEM((BS,), jnp.int32),
            pltpu.VMEM((BS * K,), jnp.int32),
        ] + [pltpu.SemaphoreType.DMA] * 11,
    )(_sc_kernel)
    out = run(page_table_dst.reshape(-1), page_table_a.reshape(-1),
              page_table_b.reshape(-1),
              seq_len_a.astype(jnp.int32), seq_len_b.astype(jnp.int32))
    return out.reshape(page_table_dst.shape)


# speedup vs baseline: 1.2182x; 1.1159x over previous
import jax
import jax.numpy as jnp
from jax.experimental import pallas as pl
from jax.experimental.pallas import tpu as pltpu

K = 4
ROWS_PER_BLK = 8
LEN_A = 4096
LEN_B = 64
LEN_DST = LEN_A + LEN_B


def _blend_kernel(la_s, lb_s, dst_ref, a_ref, b_ref, la_v, out_ref):
    i = pl.program_id(0)
    cols = jax.lax.broadcasted_iota(jnp.int32, (ROWS_PER_BLK, LEN_DST), 1)
    la = la_v[...]
    a2 = jnp.squeeze(a_ref[...], axis=1)
    a_exp = jnp.concatenate(
        [a2[0:1]] * K + [a2[1:2]] * K, axis=0)
    a_pad = jnp.concatenate(
        [a_exp, jnp.zeros((ROWS_PER_BLK, LEN_B), a_exp.dtype)], axis=1)
    out_ref[...] = jnp.where(cols < la, a_pad, dst_ref[...])

    wcols = jax.lax.broadcasted_iota(jnp.int32, (1, 128), 1)
    for r in range(ROWS_PER_BLK):
        row = i * ROWS_PER_BLK + r
        la_r = la_s[row]
        lb_r = lb_s[row]
        lab_r = la_r + lb_r
        bp = b_ref[pl.ds(r, 1), :]

        def blend(off, width):
            cols_w = wcols[:, :width] + off
            seg = out_ref[pl.ds(r, 1), pl.ds(off, width)]
            bv = pltpu.roll(bp, (la_r - off) % 128, axis=1)[:, :width]
            m_b = (cols_w >= la_r) & (cols_w < lab_r)
            out_ref[pl.ds(r, 1), pl.ds(off, width)] = jnp.where(m_b, bv, seg)

        off0 = pl.multiple_of((la_r // 128) * 128, 128)
        blend(off0, 128)

        @pl.when(lab_r > off0 + 128)
        def _():
            blend(pl.multiple_of(jnp.minimum(off0 + 128, LEN_A - 128), 128),
                  128)

        @pl.when(lab_r > LEN_A)
        def _():
            blend(LEN_A, LEN_B)


def kernel(page_table_dst, page_table_a, page_table_b, seq_len_a, seq_len_b):
    bs_expand = page_table_dst.shape[0]
    la_exp = jnp.repeat(seq_len_a.astype(jnp.int32), K)
    lb = seq_len_b.astype(jnp.int32)
    b_pad = jnp.pad(page_table_b, ((0, 0), (0, 128 - LEN_B)))
    n_blk = bs_expand // ROWS_PER_BLK
    grid_spec = pltpu.PrefetchScalarGridSpec(
        num_scalar_prefetch=2,
        grid=(n_blk,),
        in_specs=[
            pl.BlockSpec((ROWS_PER_BLK, LEN_DST), lambda i, *_: (i, 0)),
            pl.BlockSpec((ROWS_PER_BLK // K, 1, LEN_A),
                         lambda i, *_: (i, 0, 0)),
            pl.BlockSpec((ROWS_PER_BLK, 128), lambda i, *_: (i, 0)),
            pl.BlockSpec((ROWS_PER_BLK, 1), lambda i, *_: (i, 0)),
        ],
        out_specs=pl.BlockSpec((ROWS_PER_BLK, LEN_DST), lambda i, *_: (i, 0)),
    )
    return pl.pallas_call(
        _blend_kernel,
        grid_spec=grid_spec,
        out_shape=jax.ShapeDtypeStruct(page_table_dst.shape,
                                       page_table_dst.dtype),
    )(la_exp, lb, page_table_dst, page_table_a[:, None, :], b_pad,
      la_exp[:, None])


# speedup vs baseline: 1.4615x; 1.1998x over previous
import jax
import jax.numpy as jnp
from jax.experimental import pallas as pl
from jax.experimental.pallas import tpu as pltpu

K = 4
ROWS_PER_BLK = 8
LEN_A = 4096
LEN_B = 64
LEN_DST = LEN_A + LEN_B


def _blend_kernel(la_s, lb_s, dst_ref, a_ref, b_ref, la_v, out_ref):
    i = pl.program_id(0)
    cols = jax.lax.broadcasted_iota(jnp.int32, (ROWS_PER_BLK, LEN_DST), 1)
    la = la_v[...]
    a2 = jnp.squeeze(a_ref[...], axis=1)
    a_exp = jnp.concatenate(
        [a2[0:1]] * K + [a2[1:2]] * K, axis=0)
    a_pad = jnp.concatenate(
        [a_exp, jnp.zeros((ROWS_PER_BLK, LEN_B), a_exp.dtype)], axis=1)
    out_ref[...] = jnp.where(cols < la, a_pad, dst_ref[...])

    wcols = jax.lax.broadcasted_iota(jnp.int32, (1, 128), 1)
    for r in range(ROWS_PER_BLK):
        row = i * ROWS_PER_BLK + r
        la_r = la_s[row]
        lb_r = lb_s[row]
        lab_r = la_r + lb_r
        bp = b_ref[pl.ds(r, 1), :]

        def blend(off, width, use_a):
            cols_w = wcols[:, :width] + off
            seg = dst_ref[pl.ds(r, 1), pl.ds(off, width)]
            if use_a:
                a_win = a_ref[pl.ds(r // K, 1), 0, pl.ds(off, width)]
                seg = jnp.where(cols_w < la_r, a_win, seg)
            bv = pltpu.roll(bp, (la_r - off) % 128, axis=1)[:, :width]
            m_b = (cols_w >= la_r) & (cols_w < lab_r)
            out_ref[pl.ds(r, 1), pl.ds(off, width)] = jnp.where(m_b, bv, seg)

        off0 = pl.multiple_of((la_r // 128) * 128, 128)
        blend(off0, 128, True)
        blend(pl.multiple_of(jnp.minimum(off0 + 128, LEN_A - 128), 128),
              128, True)
        blend(LEN_A, LEN_B, False)


def kernel(page_table_dst, page_table_a, page_table_b, seq_len_a, seq_len_b):
    bs_expand = page_table_dst.shape[0]
    la_exp = jnp.repeat(seq_len_a.astype(jnp.int32), K)
    lb = seq_len_b.astype(jnp.int32)
    b_pad = jnp.pad(page_table_b, ((0, 0), (0, 128 - LEN_B)))
    n_blk = bs_expand // ROWS_PER_BLK
    grid_spec = pltpu.PrefetchScalarGridSpec(
        num_scalar_prefetch=2,
        grid=(n_blk,),
        in_specs=[
            pl.BlockSpec((ROWS_PER_BLK, LEN_DST), lambda i, *_: (i, 0)),
            pl.BlockSpec((ROWS_PER_BLK // K, 1, LEN_A),
                         lambda i, *_: (i, 0, 0)),
            pl.BlockSpec((ROWS_PER_BLK, 128), lambda i, *_: (i, 0)),
            pl.BlockSpec((ROWS_PER_BLK, 1), lambda i, *_: (i, 0)),
        ],
        out_specs=pl.BlockSpec((ROWS_PER_BLK, LEN_DST), lambda i, *_: (i, 0)),
    )
    return pl.pallas_call(
        _blend_kernel,
        grid_spec=grid_spec,
        out_shape=jax.ShapeDtypeStruct(page_table_dst.shape,
                                       page_table_dst.dtype),
    )(la_exp, lb, page_table_dst, page_table_a[:, None, :], b_pad,
      la_exp[:, None])


# speedup vs baseline: 1.8738x; 1.2821x over previous
import jax
import jax.numpy as jnp
from jax.experimental import pallas as pl
from jax.experimental.pallas import tpu as pltpu

K = 4
ROWS_PER_BLK = 32
LEN_A = 4096
LEN_B = 64
LEN_DST = LEN_A + LEN_B


def _blend_kernel(la_s, lb_s, dst_ref, a_ref, b_ref, la_v, out_ref):
    i = pl.program_id(0)
    cols = jax.lax.broadcasted_iota(jnp.int32, (ROWS_PER_BLK, LEN_DST), 1)
    la = la_v[...]
    a2 = jnp.squeeze(a_ref[...], axis=1)
    a_exp = jnp.concatenate(
        [a2[j:j + 1] for j in range(ROWS_PER_BLK // K) for _ in range(K)],
        axis=0)
    a_pad = jnp.concatenate(
        [a_exp, jnp.zeros((ROWS_PER_BLK, LEN_B), a_exp.dtype)], axis=1)
    out_ref[...] = jnp.where(cols < la, a_pad, dst_ref[...])

    wcols = jax.lax.broadcasted_iota(jnp.int32, (1, 128), 1)
    for r in range(ROWS_PER_BLK):
        row = i * ROWS_PER_BLK + r
        la_r = la_s[row]
        lb_r = lb_s[row]
        lab_r = la_r + lb_r
        bp = b_ref[pl.ds(r, 1), :]

        def blend(off, width, use_a):
            cols_w = wcols[:, :width] + off
            seg = dst_ref[pl.ds(r, 1), pl.ds(off, width)]
            if use_a:
                a_win = a_ref[pl.ds(r // K, 1), 0, pl.ds(off, width)]
                seg = jnp.where(cols_w < la_r, a_win, seg)
            bv = pltpu.roll(bp, (la_r - off) % 128, axis=1)[:, :width]
            m_b = (cols_w >= la_r) & (cols_w < lab_r)
            out_ref[pl.ds(r, 1), pl.ds(off, width)] = jnp.where(m_b, bv, seg)

        off0 = pl.multiple_of((la_r // 128) * 128, 128)
        blend(off0, 128, True)
        blend(pl.multiple_of(jnp.minimum(off0 + 128, LEN_A - 128), 128),
              128, True)
        blend(LEN_A, LEN_B, False)


def kernel(page_table_dst, page_table_a, page_table_b, seq_len_a, seq_len_b):
    bs_expand = page_table_dst.shape[0]
    la_exp = jnp.repeat(seq_len_a.astype(jnp.int32), K)
    lb = seq_len_b.astype(jnp.int32)
    b_pad = jnp.pad(page_table_b, ((0, 0), (0, 128 - LEN_B)))
    n_blk = bs_expand // ROWS_PER_BLK
    grid_spec = pltpu.PrefetchScalarGridSpec(
        num_scalar_prefetch=2,
        grid=(n_blk,),
        in_specs=[
            pl.BlockSpec((ROWS_PER_BLK, LEN_DST), lambda i, *_: (i, 0)),
            pl.BlockSpec((ROWS_PER_BLK // K, 1, LEN_A),
                         lambda i, *_: (i, 0, 0)),
            pl.BlockSpec((ROWS_PER_BLK, 128), lambda i, *_: (i, 0)),
            pl.BlockSpec((ROWS_PER_BLK, 1), lambda i, *_: (i, 0)),
        ],
        out_specs=pl.BlockSpec((ROWS_PER_BLK, LEN_DST), lambda i, *_: (i, 0)),
    )
    return pl.pallas_call(
        _blend_kernel,
        grid_spec=grid_spec,
        out_shape=jax.ShapeDtypeStruct(page_table_dst.shape,
                                       page_table_dst.dtype),
    )(la_exp, lb, page_table_dst, page_table_a[:, None, :], b_pad,
      la_exp[:, None])
